# Initial kernel scaffold; baseline (speedup 1.0000x reference)
import jax
import jax.numpy as jnp
from jax.experimental import pallas as pl


def kernel(x, edge_index, batch, emb_W, emb_b, gcn1_W, gcn1_b, bn1_g, bn1_b, gcn2_W, gcn2_b, bn2_g, bn2_b, fin_W, fin_b):
    raise NotImplementedError("write your pallas kernel here")


# speedup vs baseline: 9.1268x; 9.1268x over previous
import functools

import jax
import jax.numpy as jnp
from jax import lax
from jax.experimental import pallas as pl
from jax.experimental.pallas import tpu as pltpu
from jax.experimental.pallas import tpu_sc as plsc

N = 50000
E = 800000
D = 128
H = 64
HH = 32
G = 64

NSC = 2
NSUB = 16
CH = 128

N_ACC = 50176
RPS = N_ACC // NSUB
DUMP = N

E_PAD = 819200
EPS_MSG = E_PAD // NSUB
NCH_MSG = EPS_MSG // CH
EPC_DEG = E_PAD // NSC
EPS_DEG = EPC_DEG // NSUB
NCH_DEG = EPS_DEG // CH

RB = 1000
NRB = N // RB
PB = 1000
NPB = N // PB

_f32 = jnp.float32


def _sc_degree(dstp, z16, ones16):
    mesh = plsc.VectorSubcoreMesh(core_axis_name="c", subcore_axis_name="s",
                                  num_cores=NSC, num_subcores=NSUB)

    @functools.partial(
        pl.kernel,
        out_type=[jax.ShapeDtypeStruct((N_ACC, 16), _f32)] * 2,
        mesh=mesh,
        scratch_types=[
            pltpu.VMEM((CH,), jnp.int32),
            pltpu.VMEM((CH, 16), _f32),
            pltpu.VMEM_SHARED((N_ACC, 16), _f32),
        ],
        compiler_params=pltpu.CompilerParams(use_tc_tiling_on_sc=False),
    )
    def k(dst_hbm, z_hbm, ones_hbm, pa_hbm, pb_hbm, didx, ones_v, acc):
        c = lax.axis_index("c")
        s = lax.axis_index("s")
        r0 = s * RPS
        pltpu.sync_copy(z_hbm.at[pl.ds(r0, RPS)], acc.at[pl.ds(r0, RPS)])
        pltpu.sync_copy(ones_hbm, ones_v)
        plsc.subcore_barrier()

        base = c * EPC_DEG + s * EPS_DEG

        @pl.loop(0, NCH_DEG)
        def _(j):
            pltpu.sync_copy(dst_hbm.at[pl.ds(base + j * CH, CH)], didx)
            pltpu.sync_copy(ones_v, acc.at[didx], add=True)

        plsc.subcore_barrier()

        @pl.when(c == 0)
        def _():
            pltpu.sync_copy(acc.at[pl.ds(r0, RPS)], pa_hbm.at[pl.ds(r0, RPS)])

        @pl.when(c == 1)
        def _():
            pltpu.sync_copy(acc.at[pl.ds(r0, RPS)], pb_hbm.at[pl.ds(r0, RPS)])

    return k(dstp, z16, ones16)


def _sc_message(srcp, dstp, ta, tb, z32):
    mesh = plsc.VectorSubcoreMesh(core_axis_name="c", subcore_axis_name="s",
                                  num_cores=NSC, num_subcores=NSUB)

    @functools.partial(
        pl.kernel,
        out_type=[jax.ShapeDtypeStruct((N_ACC, HH), _f32)] * 2,
        mesh=mesh,
        scratch_types=[
            pltpu.VMEM((CH,), jnp.int32),
            pltpu.VMEM((CH,), jnp.int32),
            pltpu.VMEM((CH, HH), _f32),
            pltpu.VMEM_SHARED((N_ACC, HH), _f32),
        ],
        compiler_params=pltpu.CompilerParams(use_tc_tiling_on_sc=False),
    )
    def k(src_hbm, dst_hbm, ta_hbm, tb_hbm, z_hbm, oa_hbm, ob_hbm,
          sidx, didx, rows, acc):
        c = lax.axis_index("c")
        s = lax.axis_index("s")
        r0 = s * RPS
        pltpu.sync_copy(z_hbm.at[pl.ds(r0, RPS)], acc.at[pl.ds(r0, RPS)])
        plsc.subcore_barrier()

        base = s * EPS_MSG

        @pl.loop(0, NCH_MSG)
        def _(j):
            e0 = base + j * CH
            pltpu.sync_copy(src_hbm.at[pl.ds(e0, CH)], sidx)
            pltpu.sync_copy(dst_hbm.at[pl.ds(e0, CH)], didx)

            @pl.when(c == 0)
            def _():
                pltpu.sync_copy(ta_hbm.at[sidx], rows)

            @pl.when(c == 1)
            def _():
                pltpu.sync_copy(tb_hbm.at[sidx], rows)

            pltpu.sync_copy(rows, acc.at[didx], add=True)

        plsc.subcore_barrier()

        @pl.when(c == 0)
        def _():
            pltpu.sync_copy(acc.at[pl.ds(r0, RPS)], oa_hbm.at[pl.ds(r0, RPS)])

        @pl.when(c == 1)
        def _():
            pltpu.sync_copy(acc.at[pl.ds(r0, RPS)], ob_hbm.at[pl.ds(r0, RPS)])

    return k(srcp, dstp, ta, tb, z32)


def _dinv_of(p0, p1):
    deg = 1.0 + p0[:, 0:1] + p1[:, 0:1]
    return lax.rsqrt(deg)


def _tc_embed(x, emb_W, emb_b2, gcn1_W):
    def body(x_ref, w_ref, b_ref, w1_ref, hw_ref):
        h0 = jnp.dot(x_ref[...], w_ref[...],
                     preferred_element_type=_f32) + b_ref[...]
        hw_ref[...] = jnp.dot(h0, w1_ref[...], preferred_element_type=_f32)

    return pl.pallas_call(
        body,
        grid=(NRB,),
        in_specs=[
            pl.BlockSpec((RB, D), lambda i: (i, 0)),
            pl.BlockSpec((D, H), lambda i: (0, 0)),
            pl.BlockSpec((1, H), lambda i: (0, 0)),
            pl.BlockSpec((H, H), lambda i: (0, 0)),
        ],
        out_specs=pl.BlockSpec((RB, H), lambda i: (i, 0)),
        out_shape=jax.ShapeDtypeStruct((N, H), _f32),
    )(x, emb_W, emb_b2, gcn1_W)


def _tc_scale_split(hw, p0, p1):
    def body(hw_ref, p0_ref, p1_ref, ta_ref, tb_ref):
        dinv = _dinv_of(p0_ref[...], p1_ref[...])
        hws = hw_ref[...] * dinv
        ta_ref[...] = hws[:, :HH]
        tb_ref[...] = hws[:, HH:]

    return pl.pallas_call(
        body,
        grid=(NRB,),
        in_specs=[
            pl.BlockSpec((RB, H), lambda i: (i, 0)),
            pl.BlockSpec((RB, 16), lambda i: (i, 0)),
            pl.BlockSpec((RB, 16), lambda i: (i, 0)),
        ],
        out_specs=[
            pl.BlockSpec((RB, HH), lambda i: (i, 0)),
            pl.BlockSpec((RB, HH), lambda i: (i, 0)),
        ],
        out_shape=[jax.ShapeDtypeStruct((N, HH), _f32)] * 2,
    )(hw, p0, p1)


def _tc_combine_stats(agg_a, agg_b, hw, p0, p1, b2):
    def body(aa_ref, ab_ref, hw_ref, p0_ref, p1_ref, b_ref,
             z_ref, st_ref, acc_ref):
        i = pl.program_id(0)

        @pl.when(i == 0)
        def _():
            acc_ref[...] = jnp.zeros_like(acc_ref)

        dinv = _dinv_of(p0_ref[...], p1_ref[...])
        agg = jnp.concatenate([aa_ref[...], ab_ref[...]], axis=1)
        z = agg * dinv + hw_ref[...] * (dinv * dinv) + b_ref[...]
        z_ref[...] = z
        acc_ref[0:1, :] += jnp.sum(z, axis=0, keepdims=True)
        acc_ref[1:2, :] += jnp.sum(z * z, axis=0, keepdims=True)

        @pl.when(i == NRB - 1)
        def _():
            st_ref[...] = acc_ref[...]

    return pl.pallas_call(
        body,
        grid=(NRB,),
        in_specs=[
            pl.BlockSpec((RB, HH), lambda i: (i, 0)),
            pl.BlockSpec((RB, HH), lambda i: (i, 0)),
            pl.BlockSpec((RB, H), lambda i: (i, 0)),
            pl.BlockSpec((RB, 16), lambda i: (i, 0)),
            pl.BlockSpec((RB, 16), lambda i: (i, 0)),
            pl.BlockSpec((1, H), lambda i: (0, 0)),
        ],
        out_specs=[
            pl.BlockSpec((RB, H), lambda i: (i, 0)),
            pl.BlockSpec((8, H), lambda i: (0, 0)),
        ],
        out_shape=[
            jax.ShapeDtypeStruct((N, H), _f32),
            jax.ShapeDtypeStruct((8, H), _f32),
        ],
        scratch_shapes=[pltpu.VMEM((8, H), _f32)],
    )(agg_a, agg_b, hw, p0, p1, b2)


def _tc_bn_relu_matmul(z, st, g2, b2, W, p0, p1):
    def body(z_ref, st_ref, g_ref, b_ref, w_ref, p0_ref, p1_ref,
             hw_ref, ta_ref, tb_ref):
        mean = st_ref[0:1, :] * (1.0 / N)
        ex2 = st_ref[1:2, :] * (1.0 / N)
        var = ex2 - mean * mean
        inv = lax.rsqrt(var + 1e-5)
        h = jnp.maximum((z_ref[...] - mean) * inv * g_ref[...] + b_ref[...],
                        0.0)
        hw = jnp.dot(h, w_ref[...], preferred_element_type=_f32)
        hw_ref[...] = hw
        dinv = _dinv_of(p0_ref[...], p1_ref[...])
        hws = hw * dinv
        ta_ref[...] = hws[:, :HH]
        tb_ref[...] = hws[:, HH:]

    return pl.pallas_call(
        body,
        grid=(NRB,),
        in_specs=[
            pl.BlockSpec((RB, H), lambda i: (i, 0)),
            pl.BlockSpec((8, H), lambda i: (0, 0)),
            pl.BlockSpec((1, H), lambda i: (0, 0)),
            pl.BlockSpec((1, H), lambda i: (0, 0)),
            pl.BlockSpec((H, H), lambda i: (0, 0)),
            pl.BlockSpec((RB, 16), lambda i: (i, 0)),
            pl.BlockSpec((RB, 16), lambda i: (i, 0)),
        ],
        out_specs=[
            pl.BlockSpec((RB, H), lambda i: (i, 0)),
            pl.BlockSpec((RB, HH), lambda i: (i, 0)),
            pl.BlockSpec((RB, HH), lambda i: (i, 0)),
        ],
        out_shape=[
            jax.ShapeDtypeStruct((N, H), _f32),
            jax.ShapeDtypeStruct((N, HH), _f32),
            jax.ShapeDtypeStruct((N, HH), _f32),
        ],
    )(z, st, g2, b2, W, p0, p1)


def _tc_bn_relu_pool_project(z, st, g2, b2, batch3, fin_W, fin_b2):
    def body(z_ref, st_ref, g_ref, b_ref, bt_ref, fw_ref, fb_ref,
             out_ref, ssum_ref, smax_ref, cnt_ref):
        i = pl.program_id(0)

        @pl.when(i == 0)
        def _():
            ssum_ref[...] = jnp.zeros_like(ssum_ref)
            smax_ref[...] = jnp.zeros_like(smax_ref)
            cnt_ref[...] = jnp.zeros_like(cnt_ref)

        mean = st_ref[0:1, :] * (1.0 / N)
        ex2 = st_ref[1:2, :] * (1.0 / N)
        var = ex2 - mean * mean
        inv = lax.rsqrt(var + 1e-5)
        h = jnp.maximum((z_ref[...] - mean) * inv * g_ref[...] + b_ref[...],
                        0.0)

        bt = bt_ref[0, 0, :]
        onehot = (bt[:, None] ==
                  lax.broadcasted_iota(jnp.int32, (PB, G), 1)).astype(_f32)
        ssum_ref[...] += lax.dot_general(
            onehot, h, (((0,), (0,)), ((), ())), preferred_element_type=_f32)
        cnt_ref[:, 0:1] += lax.dot_general(
            onehot, jnp.ones((PB, 1), _f32), (((0,), (0,)), ((), ())),
            preferred_element_type=_f32)
        for g in range(G):
            col = onehot[:, g:g + 1]
            m = jnp.max(h * col, axis=0, keepdims=True)
            smax_ref[g:g + 1, :] = jnp.maximum(smax_ref[g:g + 1, :], m)

        @pl.when(i == NPB - 1)
        def _():
            cnt = cnt_ref[:, 0:1]
            meanp = ssum_ref[...] / jnp.maximum(cnt, 1.0)
            pooled = jnp.concatenate([smax_ref[...], meanp], axis=1)
            out_ref[...] = jnp.dot(pooled, fw_ref[...],
                                   preferred_element_type=_f32) + fb_ref[...]

    return pl.pallas_call(
        body,
        grid=(NPB,),
        in_specs=[
            pl.BlockSpec((PB, H), lambda i: (i, 0)),
            pl.BlockSpec((8, H), lambda i: (0, 0)),
            pl.BlockSpec((1, H), lambda i: (0, 0)),
            pl.BlockSpec((1, H), lambda i: (0, 0)),
            pl.BlockSpec((1, 1, PB), lambda i: (i, 0, 0)),
            pl.BlockSpec((2 * H, D), lambda i: (0, 0)),
            pl.BlockSpec((1, D), lambda i: (0, 0)),
        ],
        out_specs=pl.BlockSpec((G, D), lambda i: (0, 0)),
        out_shape=jax.ShapeDtypeStruct((G, D), _f32),
        scratch_shapes=[
            pltpu.VMEM((G, H), _f32),
            pltpu.VMEM((G, H), _f32),
            pltpu.VMEM((G, 128), _f32),
        ],
    )(z, st, g2, b2, batch3, fin_W, fin_b2)


def kernel(x, edge_index, batch, emb_W, emb_b, gcn1_W, gcn1_b, bn1_g, bn1_b,
           gcn2_W, gcn2_b, bn2_g, bn2_b, fin_W, fin_b):
    pad = E_PAD - E
    srcp = jnp.concatenate([edge_index[0], jnp.zeros((pad,), jnp.int32)])
    dstp = jnp.concatenate(
        [edge_index[1], jnp.full((pad,), DUMP, jnp.int32)])
    z16 = jnp.zeros((N_ACC, 16), _f32)
    z32 = jnp.zeros((N_ACC, HH), _f32)
    ones16 = jnp.ones((CH, 16), _f32)
    batch3 = batch.reshape(NPB, 1, PB)

    emb_b2 = emb_b.reshape(1, H)
    b1 = gcn1_b.reshape(1, H)
    b2 = gcn2_b.reshape(1, H)
    g1 = bn1_g.reshape(1, H)
    be1 = bn1_b.reshape(1, H)
    g2 = bn2_g.reshape(1, H)
    be2 = bn2_b.reshape(1, H)
    fb2 = fin_b.reshape(1, D)

    p0, p1 = _sc_degree(dstp, z16, ones16)
    hw1 = _tc_embed(x, emb_W, emb_b2, gcn1_W)

    t1a, t1b = _tc_scale_split(hw1, p0, p1)
    agg1a, agg1b = _sc_message(srcp, dstp, t1a, t1b, z32)
    z1, st1 = _tc_combine_stats(agg1a, agg1b, hw1, p0, p1, b1)

    hw2, t2a, t2b = _tc_bn_relu_matmul(z1, st1, g1, be1, gcn2_W, p0, p1)
    agg2a, agg2b = _sc_message(srcp, dstp, t2a, t2b, z32)
    z2, st2 = _tc_combine_stats(agg2a, agg2b, hw2, p0, p1, b2)

    return _tc_bn_relu_pool_project(z2, st2, g2, be2, batch3, fin_W, fb2)


# speedup vs baseline: 13.7155x; 1.5028x over previous
import functools

import jax
import jax.numpy as jnp
from jax import lax
from jax.experimental import pallas as pl
from jax.experimental.pallas import tpu as pltpu
from jax.experimental.pallas import tpu_sc as plsc

N = 50000
E = 800000
D = 128
H = 64
HH = 32
G = 64

NSC = 2
NSUB = 16
CH = 128
IW = 8

N_ACC = 50176
RPS = N_ACC // NSUB
DUMP = N

E_PAD = 819200
EPS_MSG = E_PAD // NSUB
NCH_MSG = EPS_MSG // CH
EPC_DEG = E_PAD // NSC
EPS_DEG = EPC_DEG // NSUB
NCH_DEG = EPS_DEG // CH

RB = 1000
NRB = N // RB
PB = 1000
NPB = N // PB

_f32 = jnp.float32


def _sc_degree(dstp, z16, ones16):
    mesh = plsc.VectorSubcoreMesh(core_axis_name="c", subcore_axis_name="s",
                                  num_cores=NSC, num_subcores=NSUB)

    @functools.partial(
        pl.kernel,
        out_type=[jax.ShapeDtypeStruct((N_ACC, 16), _f32)] * 2,
        mesh=mesh,
        scratch_types=[
            pltpu.VMEM((CH,), jnp.int32),
            pltpu.VMEM((CH, 16), _f32),
            pltpu.VMEM_SHARED((N_ACC, 16), _f32),
        ],
        compiler_params=pltpu.CompilerParams(use_tc_tiling_on_sc=False),
    )
    def k(dst_hbm, z_hbm, ones_hbm, pa_hbm, pb_hbm, didx, ones_v, acc):
        c = lax.axis_index("c")
        s = lax.axis_index("s")
        r0 = s * RPS
        pltpu.sync_copy(z_hbm.at[pl.ds(r0, RPS)], acc.at[pl.ds(r0, RPS)])
        pltpu.sync_copy(ones_hbm, ones_v)
        plsc.subcore_barrier()

        base = c * EPC_DEG + s * EPS_DEG

        @pl.loop(0, NCH_DEG)
        def _(j):
            pltpu.sync_copy(dst_hbm.at[pl.ds(base + j * CH, CH)], didx)
            pltpu.sync_copy(ones_v, acc.at[didx], add=True)

        plsc.subcore_barrier()

        @pl.when(c == 0)
        def _():
            pltpu.sync_copy(acc.at[pl.ds(r0, RPS)], pa_hbm.at[pl.ds(r0, RPS)])

        @pl.when(c == 1)
        def _():
            pltpu.sync_copy(acc.at[pl.ds(r0, RPS)], pb_hbm.at[pl.ds(r0, RPS)])

    return k(dstp, z16, ones16)


def _sc_message(srcp, dstp, ta, tb, z32):
    mesh = plsc.VectorSubcoreMesh(core_axis_name="c", subcore_axis_name="s",
                                  num_cores=NSC, num_subcores=NSUB)

    @functools.partial(
        pl.kernel,
        out_type=[jax.ShapeDtypeStruct((N_ACC, HH), _f32)] * 2,
        mesh=mesh,
        scratch_types=[
            pltpu.VMEM((IW, CH), jnp.int32),
            pltpu.VMEM((IW, CH), jnp.int32),
            pltpu.VMEM((2, CH, HH), _f32),
            pltpu.VMEM_SHARED((N_ACC, HH), _f32),
            pltpu.SemaphoreType.DMA,
            pltpu.SemaphoreType.DMA,
            pltpu.SemaphoreType.DMA,
            pltpu.SemaphoreType.DMA,
        ],
        compiler_params=pltpu.CompilerParams(use_tc_tiling_on_sc=False),
    )
    def k(src_hbm, dst_hbm, ta_hbm, tb_hbm, z_hbm, oa_hbm, ob_hbm,
          sidx, didx, rows, acc, gs0, gs1, ss0, ss1):
        c = lax.axis_index("c")
        s = lax.axis_index("s")
        r0 = s * RPS
        pltpu.sync_copy(z_hbm.at[pl.ds(r0, RPS)], acc.at[pl.ds(r0, RPS)])
        plsc.subcore_barrier()

        gsem = (gs0, gs1)
        ssem = (ss0, ss1)

        def start_gather(jj, buf, sem):
            @pl.when(c == 0)
            def _():
                pltpu.async_copy(ta_hbm.at[sidx.at[jj]], buf, sem)

            @pl.when(c == 1)
            def _():
                pltpu.async_copy(tb_hbm.at[sidx.at[jj]], buf, sem)

        def drain(buf, sem):
            pltpu.make_async_copy(z_hbm.at[pl.ds(0, CH)], buf, sem).wait()

        sbase8 = s * (EPS_MSG // CH)

        @pl.loop(0, NCH_MSG // IW)
        def _(o):
            ro = sbase8 + o * IW

            @pl.when(o > 0)
            def _():
                drain(rows.at[0], ss0)
                drain(rows.at[1], ss1)

            pltpu.sync_copy(src_hbm.at[pl.ds(ro, IW)], sidx)
            pltpu.sync_copy(dst_hbm.at[pl.ds(ro, IW)], didx)

            start_gather(0, rows.at[0], gs0)
            for j in range(IW):
                b = j & 1
                nb = 1 - b
                if j + 1 < IW:
                    if j >= 1:
                        drain(rows.at[nb], ssem[nb])
                    start_gather(j + 1, rows.at[nb], gsem[nb])
                drain(rows.at[b], gsem[b])
                pltpu.async_copy(rows.at[b], acc.at[didx.at[j]], ssem[b],
                                 add=True)

        drain(rows.at[0], ss0)
        drain(rows.at[1], ss1)
        plsc.subcore_barrier()

        @pl.when(c == 0)
        def _():
            pltpu.sync_copy(acc.at[pl.ds(r0, RPS)], oa_hbm.at[pl.ds(r0, RPS)])

        @pl.when(c == 1)
        def _():
            pltpu.sync_copy(acc.at[pl.ds(r0, RPS)], ob_hbm.at[pl.ds(r0, RPS)])

    return k(srcp.reshape(-1, CH), dstp.reshape(-1, CH), ta, tb, z32)


def _dinv_of(p0, p1):
    deg = 1.0 + p0[:, 0:1] + p1[:, 0:1]
    return lax.rsqrt(deg)


def _tc_embed(x, emb_W, emb_b2, gcn1_W):
    def body(x_ref, w_ref, b_ref, w1_ref, hw_ref):
        h0 = jnp.dot(x_ref[...], w_ref[...],
                     preferred_element_type=_f32) + b_ref[...]
        hw_ref[...] = jnp.dot(h0, w1_ref[...], preferred_element_type=_f32)

    return pl.pallas_call(
        body,
        grid=(NRB,),
        in_specs=[
            pl.BlockSpec((RB, D), lambda i: (i, 0)),
            pl.BlockSpec((D, H), lambda i: (0, 0)),
            pl.BlockSpec((1, H), lambda i: (0, 0)),
            pl.BlockSpec((H, H), lambda i: (0, 0)),
        ],
        out_specs=pl.BlockSpec((RB, H), lambda i: (i, 0)),
        out_shape=jax.ShapeDtypeStruct((N, H), _f32),
    )(x, emb_W, emb_b2, gcn1_W)


def _tc_scale_split(hw, p0, p1):
    def body(hw_ref, p0_ref, p1_ref, ta_ref, tb_ref):
        dinv = _dinv_of(p0_ref[...], p1_ref[...])
        hws = hw_ref[...] * dinv
        ta_ref[...] = hws[:, :HH]
        tb_ref[...] = hws[:, HH:]

    return pl.pallas_call(
        body,
        grid=(NRB,),
        in_specs=[
            pl.BlockSpec((RB, H), lambda i: (i, 0)),
            pl.BlockSpec((RB, 16), lambda i: (i, 0)),
            pl.BlockSpec((RB, 16), lambda i: (i, 0)),
        ],
        out_specs=[
            pl.BlockSpec((RB, HH), lambda i: (i, 0)),
            pl.BlockSpec((RB, HH), lambda i: (i, 0)),
        ],
        out_shape=[jax.ShapeDtypeStruct((N, HH), _f32)] * 2,
    )(hw, p0, p1)


def _tc_combine_stats(agg_a, agg_b, hw, p0, p1, b2):
    def body(aa_ref, ab_ref, hw_ref, p0_ref, p1_ref, b_ref,
             z_ref, st_ref, acc_ref):
        i = pl.program_id(0)

        @pl.when(i == 0)
        def _():
            acc_ref[...] = jnp.zeros_like(acc_ref)

        dinv = _dinv_of(p0_ref[...], p1_ref[...])
        agg = jnp.concatenate([aa_ref[...], ab_ref[...]], axis=1)
        z = agg * dinv + hw_ref[...] * (dinv * dinv) + b_ref[...]
        z_ref[...] = z
        acc_ref[0:1, :] += jnp.sum(z, axis=0, keepdims=True)
        acc_ref[1:2, :] += jnp.sum(z * z, axis=0, keepdims=True)

        @pl.when(i == NRB - 1)
        def _():
            st_ref[...] = acc_ref[...]

    return pl.pallas_call(
        body,
        grid=(NRB,),
        in_specs=[
            pl.BlockSpec((RB, HH), lambda i: (i, 0)),
            pl.BlockSpec((RB, HH), lambda i: (i, 0)),
            pl.BlockSpec((RB, H), lambda i: (i, 0)),
            pl.BlockSpec((RB, 16), lambda i: (i, 0)),
            pl.BlockSpec((RB, 16), lambda i: (i, 0)),
            pl.BlockSpec((1, H), lambda i: (0, 0)),
        ],
        out_specs=[
            pl.BlockSpec((RB, H), lambda i: (i, 0)),
            pl.BlockSpec((8, H), lambda i: (0, 0)),
        ],
        out_shape=[
            jax.ShapeDtypeStruct((N, H), _f32),
            jax.ShapeDtypeStruct((8, H), _f32),
        ],
        scratch_shapes=[pltpu.VMEM((8, H), _f32)],
    )(agg_a, agg_b, hw, p0, p1, b2)


def _tc_bn_relu_matmul(z, st, g2, b2, W, p0, p1):
    def body(z_ref, st_ref, g_ref, b_ref, w_ref, p0_ref, p1_ref,
             hw_ref, ta_ref, tb_ref):
        mean = st_ref[0:1, :] * (1.0 / N)
        ex2 = st_ref[1:2, :] * (1.0 / N)
        var = ex2 - mean * mean
        inv = lax.rsqrt(var + 1e-5)
        h = jnp.maximum((z_ref[...] - mean) * inv * g_ref[...] + b_ref[...],
                        0.0)
        hw = jnp.dot(h, w_ref[...], preferred_element_type=_f32)
        hw_ref[...] = hw
        dinv = _dinv_of(p0_ref[...], p1_ref[...])
        hws = hw * dinv
        ta_ref[...] = hws[:, :HH]
        tb_ref[...] = hws[:, HH:]

    return pl.pallas_call(
        body,
        grid=(NRB,),
        in_specs=[
            pl.BlockSpec((RB, H), lambda i: (i, 0)),
            pl.BlockSpec((8, H), lambda i: (0, 0)),
            pl.BlockSpec((1, H), lambda i: (0, 0)),
            pl.BlockSpec((1, H), lambda i: (0, 0)),
            pl.BlockSpec((H, H), lambda i: (0, 0)),
            pl.BlockSpec((RB, 16), lambda i: (i, 0)),
            pl.BlockSpec((RB, 16), lambda i: (i, 0)),
        ],
        out_specs=[
            pl.BlockSpec((RB, H), lambda i: (i, 0)),
            pl.BlockSpec((RB, HH), lambda i: (i, 0)),
            pl.BlockSpec((RB, HH), lambda i: (i, 0)),
        ],
        out_shape=[
            jax.ShapeDtypeStruct((N, H), _f32),
            jax.ShapeDtypeStruct((N, HH), _f32),
            jax.ShapeDtypeStruct((N, HH), _f32),
        ],
    )(z, st, g2, b2, W, p0, p1)


def _tc_bn_relu_pool_project(z, st, g2, b2, batch3, fin_W, fin_b2):
    def body(z_ref, st_ref, g_ref, b_ref, bt_ref, fw_ref, fb_ref,
             out_ref, ssum_ref, smax_ref, cnt_ref):
        i = pl.program_id(0)

        @pl.when(i == 0)
        def _():
            ssum_ref[...] = jnp.zeros_like(ssum_ref)
            smax_ref[...] = jnp.zeros_like(smax_ref)
            cnt_ref[...] = jnp.zeros_like(cnt_ref)

        mean = st_ref[0:1, :] * (1.0 / N)
        ex2 = st_ref[1:2, :] * (1.0 / N)
        var = ex2 - mean * mean
        inv = lax.rsqrt(var + 1e-5)
        h = jnp.maximum((z_ref[...] - mean) * inv * g_ref[...] + b_ref[...],
                        0.0)

        bt = bt_ref[0, 0, :]
        onehot = (bt[:, None] ==
                  lax.broadcasted_iota(jnp.int32, (PB, G), 1)).astype(_f32)
        ssum_ref[...] += lax.dot_general(
            onehot, h, (((0,), (0,)), ((), ())), preferred_element_type=_f32)
        cnt_ref[:, 0:1] += lax.dot_general(
            onehot, jnp.ones((PB, 1), _f32), (((0,), (0,)), ((), ())),
            preferred_element_type=_f32)
        for g in range(G):
            col = onehot[:, g:g + 1]
            m = jnp.max(h * col, axis=0, keepdims=True)
            smax_ref[g:g + 1, :] = jnp.maximum(smax_ref[g:g + 1, :], m)

        @pl.when(i == NPB - 1)
        def _():
            cnt = cnt_ref[:, 0:1]
            meanp = ssum_ref[...] / jnp.maximum(cnt, 1.0)
            pooled = jnp.concatenate([smax_ref[...], meanp], axis=1)
            out_ref[...] = jnp.dot(pooled, fw_ref[...],
                                   preferred_element_type=_f32) + fb_ref[...]

    return pl.pallas_call(
        body,
        grid=(NPB,),
        in_specs=[
            pl.BlockSpec((PB, H), lambda i: (i, 0)),
            pl.BlockSpec((8, H), lambda i: (0, 0)),
            pl.BlockSpec((1, H), lambda i: (0, 0)),
            pl.BlockSpec((1, H), lambda i: (0, 0)),
            pl.BlockSpec((1, 1, PB), lambda i: (i, 0, 0)),
            pl.BlockSpec((2 * H, D), lambda i: (0, 0)),
            pl.BlockSpec((1, D), lambda i: (0, 0)),
        ],
        out_specs=pl.BlockSpec((G, D), lambda i: (0, 0)),
        out_shape=jax.ShapeDtypeStruct((G, D), _f32),
        scratch_shapes=[
            pltpu.VMEM((G, H), _f32),
            pltpu.VMEM((G, H), _f32),
            pltpu.VMEM((G, 128), _f32),
        ],
    )(z, st, g2, b2, batch3, fin_W, fin_b2)


def kernel(x, edge_index, batch, emb_W, emb_b, gcn1_W, gcn1_b, bn1_g, bn1_b,
           gcn2_W, gcn2_b, bn2_g, bn2_b, fin_W, fin_b):
    pad = E_PAD - E
    srcp = jnp.concatenate([edge_index[0], jnp.zeros((pad,), jnp.int32)])
    dstp = jnp.concatenate(
        [edge_index[1], jnp.full((pad,), DUMP, jnp.int32)])
    z16 = jnp.zeros((N_ACC, 16), _f32)
    z32 = jnp.zeros((N_ACC, HH), _f32)
    ones16 = jnp.ones((CH, 16), _f32)
    batch3 = batch.reshape(NPB, 1, PB)

    emb_b2 = emb_b.reshape(1, H)
    b1 = gcn1_b.reshape(1, H)
    b2 = gcn2_b.reshape(1, H)
    g1 = bn1_g.reshape(1, H)
    be1 = bn1_b.reshape(1, H)
    g2 = bn2_g.reshape(1, H)
    be2 = bn2_b.reshape(1, H)
    fb2 = fin_b.reshape(1, D)

    p0, p1 = _sc_degree(dstp, z16, ones16)
    hw1 = _tc_embed(x, emb_W, emb_b2, gcn1_W)

    t1a, t1b = _tc_scale_split(hw1, p0, p1)
    agg1a, agg1b = _sc_message(srcp, dstp, t1a, t1b, z32)
    z1, st1 = _tc_combine_stats(agg1a, agg1b, hw1, p0, p1, b1)

    hw2, t2a, t2b = _tc_bn_relu_matmul(z1, st1, g1, be1, gcn2_W, p0, p1)
    agg2a, agg2b = _sc_message(srcp, dstp, t2a, t2b, z32)
    z2, st2 = _tc_combine_stats(agg2a, agg2b, hw2, p0, p1, b2)

    return _tc_bn_relu_pool_project(z2, st2, g2, be2, batch3, fin_W, fb2)


# speedup vs baseline: 15.3144x; 1.1166x over previous
import functools

import jax
import jax.numpy as jnp
from jax import lax
from jax.experimental import pallas as pl
from jax.experimental.pallas import tpu as pltpu
from jax.experimental.pallas import tpu_sc as plsc

N = 50000
E = 800000
D = 128
H = 64
HH = 32
G = 64

NSC = 2
NSUB = 16
CH = 128
IW = 10

N_ACC = 50176
RPS = N_ACC // NSUB
DUMP = N

E_PAD = 819200
EPS_MSG = E_PAD // NSUB
NCH_MSG = EPS_MSG // CH
EPC_DEG = E_PAD // NSC
EPS_DEG = EPC_DEG // NSUB
NCH_DEG = EPS_DEG // CH

RB = 1000
NRB = N // RB
PB = 1000
NPB = N // PB

_f32 = jnp.float32


def _sc_degree(dstp, z16, ones16):
    mesh = plsc.VectorSubcoreMesh(core_axis_name="c", subcore_axis_name="s",
                                  num_cores=NSC, num_subcores=NSUB)

    @functools.partial(
        pl.kernel,
        out_type=[jax.ShapeDtypeStruct((N_ACC, 16), _f32)] * 2,
        mesh=mesh,
        scratch_types=[
            pltpu.VMEM((2, IW, CH), jnp.int32),
            pltpu.VMEM((CH, 16), _f32),
            pltpu.VMEM_SHARED((N_ACC, 16), _f32),
            pltpu.SemaphoreType.DMA,
            pltpu.SemaphoreType.DMA,
            pltpu.SemaphoreType.DMA,
            pltpu.SemaphoreType.DMA,
        ],
        compiler_params=pltpu.CompilerParams(use_tc_tiling_on_sc=False),
    )
    def k(dst_hbm, z_hbm, ones_hbm, pa_hbm, pb_hbm, didx, ones_v, acc,
          is0, is1, ss0, ss1):
        c = lax.axis_index("c")
        s = lax.axis_index("s")
        r0 = s * RPS
        pltpu.sync_copy(z_hbm.at[pl.ds(r0, RPS)], acc.at[pl.ds(r0, RPS)])
        pltpu.sync_copy(ones_hbm, ones_v)
        plsc.subcore_barrier()

        cbase = (c * EPC_DEG + s * EPS_DEG) // CH
        NO = NCH_DEG // IW

        def idx_drain(b, sem):
            pltpu.make_async_copy(dst_hbm.at[pl.ds(0, IW)], didx.at[b],
                                  sem).wait()

        def scat_drain(sem):
            pltpu.make_async_copy(z_hbm.at[pl.ds(0, CH)], ones_v, sem).wait()

        pltpu.async_copy(dst_hbm.at[pl.ds(cbase, IW)], didx.at[0], is0)

        @pl.loop(0, NO // 2)
        def _(o2):
            o = o2 * 2
            @pl.when(o2 > 0)
            def _():
                for _ in range(IW):
                    scat_drain(ss1)

            pltpu.async_copy(dst_hbm.at[pl.ds(cbase + (o + 1) * IW, IW)],
                             didx.at[1], is1)
            idx_drain(0, is0)
            for j in range(IW):
                pltpu.async_copy(ones_v, acc.at[didx.at[0].at[j]], ss0,
                                 add=True)

            for _ in range(IW):
                scat_drain(ss0)

            @pl.when(o2 + 1 < NO // 2)
            def _():
                pltpu.async_copy(dst_hbm.at[pl.ds(cbase + (o + 2) * IW, IW)],
                                 didx.at[0], is0)

            idx_drain(1, is1)
            for j in range(IW):
                pltpu.async_copy(ones_v, acc.at[didx.at[1].at[j]], ss1,
                                 add=True)

        for _ in range(IW):
            scat_drain(ss1)
        plsc.subcore_barrier()

        @pl.when(c == 0)
        def _():
            pltpu.sync_copy(acc.at[pl.ds(r0, RPS)], pa_hbm.at[pl.ds(r0, RPS)])

        @pl.when(c == 1)
        def _():
            pltpu.sync_copy(acc.at[pl.ds(r0, RPS)], pb_hbm.at[pl.ds(r0, RPS)])

    return k(dstp.reshape(-1, CH), z16, ones16)


def _sc_message(srcp, dstp, ta, tb, z32):
    mesh = plsc.VectorSubcoreMesh(core_axis_name="c", subcore_axis_name="s",
                                  num_cores=NSC, num_subcores=NSUB)

    @functools.partial(
        pl.kernel,
        out_type=[jax.ShapeDtypeStruct((N_ACC, HH), _f32)] * 2,
        mesh=mesh,
        scratch_types=[
            pltpu.VMEM((IW, CH), jnp.int32),
            pltpu.VMEM((IW, CH), jnp.int32),
            pltpu.VMEM((4, CH, HH), _f32),
            pltpu.VMEM_SHARED((N_ACC, HH), _f32),
            pltpu.SemaphoreType.DMA,
            pltpu.SemaphoreType.DMA,
            pltpu.SemaphoreType.DMA,
            pltpu.SemaphoreType.DMA,
            pltpu.SemaphoreType.DMA,
            pltpu.SemaphoreType.DMA,
            pltpu.SemaphoreType.DMA,
            pltpu.SemaphoreType.DMA,
        ],
        compiler_params=pltpu.CompilerParams(use_tc_tiling_on_sc=False),
    )
    def k(src_hbm, dst_hbm, ta_hbm, tb_hbm, z_hbm, oa_hbm, ob_hbm,
          sidx, didx, rows, acc, gs0, gs1, gs2, gs3, ss0, ss1, ss2, ss3):
        c = lax.axis_index("c")
        s = lax.axis_index("s")
        r0 = s * RPS
        pltpu.sync_copy(z_hbm.at[pl.ds(r0, RPS)], acc.at[pl.ds(r0, RPS)])
        plsc.subcore_barrier()

        gsem = (gs0, gs1, gs2, gs3)
        ssem = (ss0, ss1, ss2, ss3)

        def start_gather(jj, buf, sem):
            @pl.when(c == 0)
            def _():
                pltpu.async_copy(ta_hbm.at[sidx.at[jj]], buf, sem)

            @pl.when(c == 1)
            def _():
                pltpu.async_copy(tb_hbm.at[sidx.at[jj]], buf, sem)

        def drain(buf, sem):
            pltpu.make_async_copy(z_hbm.at[pl.ds(0, CH)], buf, sem).wait()

        sbase8 = s * (EPS_MSG // CH)

        @pl.loop(0, NCH_MSG // IW)
        def _(o):
            ro = sbase8 + o * IW

            @pl.when(o > 0)
            def _():
                for b in range(4):
                    drain(rows.at[b], ssem[b])

            pltpu.sync_copy(src_hbm.at[pl.ds(ro, IW)], sidx)
            pltpu.sync_copy(dst_hbm.at[pl.ds(ro, IW)], didx)

            start_gather(0, rows.at[0], gs0)
            start_gather(1, rows.at[1], gs1)
            for j in range(IW):
                b = j % 4
                if j + 2 < IW:
                    fb = (j + 2) % 4
                    if j >= 2:
                        drain(rows.at[fb], ssem[fb])
                    start_gather(j + 2, rows.at[fb], gsem[fb])
                drain(rows.at[b], gsem[b])
                pltpu.async_copy(rows.at[b], acc.at[didx.at[j]], ssem[b],
                                 add=True)

        for b in range(4):
            drain(rows.at[b], ssem[b])
        plsc.subcore_barrier()

        @pl.when(c == 0)
        def _():
            pltpu.sync_copy(acc.at[pl.ds(r0, RPS)], oa_hbm.at[pl.ds(r0, RPS)])

        @pl.when(c == 1)
        def _():
            pltpu.sync_copy(acc.at[pl.ds(r0, RPS)], ob_hbm.at[pl.ds(r0, RPS)])

    return k(srcp.reshape(-1, CH), dstp.reshape(-1, CH), ta, tb, z32)


def _dinv_of(p0, p1):
    deg = 1.0 + p0[:, 0:1] + p1[:, 0:1]
    return lax.rsqrt(deg)


def _tc_embed(x, emb_W, emb_b2, gcn1_W):
    def body(x_ref, w_ref, b_ref, w1_ref, hw_ref):
        h0 = jnp.dot(x_ref[...], w_ref[...],
                     preferred_element_type=_f32) + b_ref[...]
        hw_ref[...] = jnp.dot(h0, w1_ref[...], preferred_element_type=_f32)

    return pl.pallas_call(
        body,
        grid=(NRB,),
        in_specs=[
            pl.BlockSpec((RB, D), lambda i: (i, 0)),
            pl.BlockSpec((D, H), lambda i: (0, 0)),
            pl.BlockSpec((1, H), lambda i: (0, 0)),
            pl.BlockSpec((H, H), lambda i: (0, 0)),
        ],
        out_specs=pl.BlockSpec((RB, H), lambda i: (i, 0)),
        out_shape=jax.ShapeDtypeStruct((N, H), _f32),
    )(x, emb_W, emb_b2, gcn1_W)


def _tc_scale_split(hw, p0, p1):
    def body(hw_ref, p0_ref, p1_ref, ta_ref, tb_ref):
        dinv = _dinv_of(p0_ref[...], p1_ref[...])
        hws = hw_ref[...] * dinv
        ta_ref[...] = hws[:, :HH]
        tb_ref[...] = hws[:, HH:]

    return pl.pallas_call(
        body,
        grid=(NRB,),
        in_specs=[
            pl.BlockSpec((RB, H), lambda i: (i, 0)),
            pl.BlockSpec((RB, 16), lambda i: (i, 0)),
            pl.BlockSpec((RB, 16), lambda i: (i, 0)),
        ],
        out_specs=[
            pl.BlockSpec((RB, HH), lambda i: (i, 0)),
            pl.BlockSpec((RB, HH), lambda i: (i, 0)),
        ],
        out_shape=[jax.ShapeDtypeStruct((N, HH), _f32)] * 2,
    )(hw, p0, p1)


def _tc_combine_stats(agg_a, agg_b, hw, p0, p1, b2):
    def body(aa_ref, ab_ref, hw_ref, p0_ref, p1_ref, b_ref,
             z_ref, st_ref, acc_ref):
        i = pl.program_id(0)

        @pl.when(i == 0)
        def _():
            acc_ref[...] = jnp.zeros_like(acc_ref)

        dinv = _dinv_of(p0_ref[...], p1_ref[...])
        agg = jnp.concatenate([aa_ref[...], ab_ref[...]], axis=1)
        z = agg * dinv + hw_ref[...] * (dinv * dinv) + b_ref[...]
        z_ref[...] = z
        acc_ref[0:1, :] += jnp.sum(z, axis=0, keepdims=True)
        acc_ref[1:2, :] += jnp.sum(z * z, axis=0, keepdims=True)

        @pl.when(i == NRB - 1)
        def _():
            st_ref[...] = acc_ref[...]

    return pl.pallas_call(
        body,
        grid=(NRB,),
        in_specs=[
            pl.BlockSpec((RB, HH), lambda i: (i, 0)),
            pl.BlockSpec((RB, HH), lambda i: (i, 0)),
            pl.BlockSpec((RB, H), lambda i: (i, 0)),
            pl.BlockSpec((RB, 16), lambda i: (i, 0)),
            pl.BlockSpec((RB, 16), lambda i: (i, 0)),
            pl.BlockSpec((1, H), lambda i: (0, 0)),
        ],
        out_specs=[
            pl.BlockSpec((RB, H), lambda i: (i, 0)),
            pl.BlockSpec((8, H), lambda i: (0, 0)),
        ],
        out_shape=[
            jax.ShapeDtypeStruct((N, H), _f32),
            jax.ShapeDtypeStruct((8, H), _f32),
        ],
        scratch_shapes=[pltpu.VMEM((8, H), _f32)],
    )(agg_a, agg_b, hw, p0, p1, b2)


def _tc_bn_relu_matmul(z, st, g2, b2, W, p0, p1):
    def body(z_ref, st_ref, g_ref, b_ref, w_ref, p0_ref, p1_ref,
             hw_ref, ta_ref, tb_ref):
        mean = st_ref[0:1, :] * (1.0 / N)
        ex2 = st_ref[1:2, :] * (1.0 / N)
        var = ex2 - mean * mean
        inv = lax.rsqrt(var + 1e-5)
        h = jnp.maximum((z_ref[...] - mean) * inv * g_ref[...] + b_ref[...],
                        0.0)
        hw = jnp.dot(h, w_ref[...], preferred_element_type=_f32)
        hw_ref[...] = hw
        dinv = _dinv_of(p0_ref[...], p1_ref[...])
        hws = hw * dinv
        ta_ref[...] = hws[:, :HH]
        tb_ref[...] = hws[:, HH:]

    return pl.pallas_call(
        body,
        grid=(NRB,),
        in_specs=[
            pl.BlockSpec((RB, H), lambda i: (i, 0)),
            pl.BlockSpec((8, H), lambda i: (0, 0)),
            pl.BlockSpec((1, H), lambda i: (0, 0)),
            pl.BlockSpec((1, H), lambda i: (0, 0)),
            pl.BlockSpec((H, H), lambda i: (0, 0)),
            pl.BlockSpec((RB, 16), lambda i: (i, 0)),
            pl.BlockSpec((RB, 16), lambda i: (i, 0)),
        ],
        out_specs=[
            pl.BlockSpec((RB, H), lambda i: (i, 0)),
            pl.BlockSpec((RB, HH), lambda i: (i, 0)),
            pl.BlockSpec((RB, HH), lambda i: (i, 0)),
        ],
        out_shape=[
            jax.ShapeDtypeStruct((N, H), _f32),
            jax.ShapeDtypeStruct((N, HH), _f32),
            jax.ShapeDtypeStruct((N, HH), _f32),
        ],
    )(z, st, g2, b2, W, p0, p1)


def _tc_bn_relu_pool_project(z, st, g2, b2, batch3, fin_W, fin_b2):
    def body(z_ref, st_ref, g_ref, b_ref, bt_ref, fw_ref, fb_ref,
             out_ref, ssum_ref, smax_ref, cnt_ref):
        i = pl.program_id(0)

        @pl.when(i == 0)
        def _():
            ssum_ref[...] = jnp.zeros_like(ssum_ref)
            smax_ref[...] = jnp.zeros_like(smax_ref)
            cnt_ref[...] = jnp.zeros_like(cnt_ref)

        mean = st_ref[0:1, :] * (1.0 / N)
        ex2 = st_ref[1:2, :] * (1.0 / N)
        var = ex2 - mean * mean
        inv = lax.rsqrt(var + 1e-5)
        h = jnp.maximum((z_ref[...] - mean) * inv * g_ref[...] + b_ref[...],
                        0.0)

        bt = bt_ref[0, 0, :]
        onehot = (bt[:, None] ==
                  lax.broadcasted_iota(jnp.int32, (PB, G), 1)).astype(_f32)
        ssum_ref[...] += lax.dot_general(
            onehot, h, (((0,), (0,)), ((), ())), preferred_element_type=_f32)
        cnt_ref[:, 0:1] += lax.dot_general(
            onehot, jnp.ones((PB, 1), _f32), (((0,), (0,)), ((), ())),
            preferred_element_type=_f32)
        for g in range(G):
            col = onehot[:, g:g + 1]
            m = jnp.max(h * col, axis=0, keepdims=True)
            smax_ref[g:g + 1, :] = jnp.maximum(smax_ref[g:g + 1, :], m)

        @pl.when(i == NPB - 1)
        def _():
            cnt = cnt_ref[:, 0:1]
            meanp = ssum_ref[...] / jnp.maximum(cnt, 1.0)
            pooled = jnp.concatenate([smax_ref[...], meanp], axis=1)
            out_ref[...] = jnp.dot(pooled, fw_ref[...],
                                   preferred_element_type=_f32) + fb_ref[...]

    return pl.pallas_call(
        body,
        grid=(NPB,),
        in_specs=[
            pl.BlockSpec((PB, H), lambda i: (i, 0)),
            pl.BlockSpec((8, H), lambda i: (0, 0)),
            pl.BlockSpec((1, H), lambda i: (0, 0)),
            pl.BlockSpec((1, H), lambda i: (0, 0)),
            pl.BlockSpec((1, 1, PB), lambda i: (i, 0, 0)),
            pl.BlockSpec((2 * H, D), lambda i: (0, 0)),
            pl.BlockSpec((1, D), lambda i: (0, 0)),
        ],
        out_specs=pl.BlockSpec((G, D), lambda i: (0, 0)),
        out_shape=jax.ShapeDtypeStruct((G, D), _f32),
        scratch_shapes=[
            pltpu.VMEM((G, H), _f32),
            pltpu.VMEM((G, H), _f32),
            pltpu.VMEM((G, 128), _f32),
        ],
    )(z, st, g2, b2, batch3, fin_W, fin_b2)


def kernel(x, edge_index, batch, emb_W, emb_b, gcn1_W, gcn1_b, bn1_g, bn1_b,
           gcn2_W, gcn2_b, bn2_g, bn2_b, fin_W, fin_b):
    pad = E_PAD - E
    srcp = jnp.concatenate([edge_index[0], jnp.zeros((pad,), jnp.int32)])
    dstp = jnp.concatenate(
        [edge_index[1], jnp.full((pad,), DUMP, jnp.int32)])
    z16 = jnp.zeros((N_ACC, 16), _f32)
    z32 = jnp.zeros((N_ACC, HH), _f32)
    ones16 = jnp.ones((CH, 16), _f32)
    batch3 = batch.reshape(NPB, 1, PB)

    emb_b2 = emb_b.reshape(1, H)
    b1 = gcn1_b.reshape(1, H)
    b2 = gcn2_b.reshape(1, H)
    g1 = bn1_g.reshape(1, H)
    be1 = bn1_b.reshape(1, H)
    g2 = bn2_g.reshape(1, H)
    be2 = bn2_b.reshape(1, H)
    fb2 = fin_b.reshape(1, D)

    p0, p1 = _sc_degree(dstp, z16, ones16)
    hw1 = _tc_embed(x, emb_W, emb_b2, gcn1_W)

    t1a, t1b = _tc_scale_split(hw1, p0, p1)
    agg1a, agg1b = _sc_message(srcp, dstp, t1a, t1b, z32)
    z1, st1 = _tc_combine_stats(agg1a, agg1b, hw1, p0, p1, b1)

    hw2, t2a, t2b = _tc_bn_relu_matmul(z1, st1, g1, be1, gcn2_W, p0, p1)
    agg2a, agg2b = _sc_message(srcp, dstp, t2a, t2b, z32)
    z2, st2 = _tc_combine_stats(agg2a, agg2b, hw2, p0, p1, b2)

    return _tc_bn_relu_pool_project(z2, st2, g2, be2, batch3, fin_W, fb2)


# speedup vs baseline: 17.4050x; 1.1365x over previous
import functools

import jax
import jax.numpy as jnp
from jax import lax
from jax.experimental import pallas as pl
from jax.experimental.pallas import tpu as pltpu
from jax.experimental.pallas import tpu_sc as plsc

N = 50000
E = 800000
D = 128
H = 64
HH = 32
G = 64

NSC = 2
NSUB = 16
CH = 128
IW = 10

N_ACC = 50176
RPS = N_ACC // NSUB
DUMP = N

E_PAD = 819200
EPS_MSG = E_PAD // NSUB
NCH_MSG = EPS_MSG // CH
EPC_DEG = E_PAD // NSC
EPS_DEG = EPC_DEG // NSUB
NCH_DEG = EPS_DEG // CH

RB = 1000
NRB = N // RB
PB = 1000
NPB = N // PB

_f32 = jnp.float32


def _sc_degree(dstp, z16, ones16):
    mesh = plsc.VectorSubcoreMesh(core_axis_name="c", subcore_axis_name="s",
                                  num_cores=NSC, num_subcores=NSUB)

    @functools.partial(
        pl.kernel,
        out_type=[jax.ShapeDtypeStruct((N_ACC, 16), _f32)] * 2,
        mesh=mesh,
        scratch_types=[
            pltpu.VMEM((2, IW, CH), jnp.int32),
            pltpu.VMEM((CH, 16), _f32),
            pltpu.VMEM_SHARED((N_ACC, 16), _f32),
            pltpu.SemaphoreType.DMA,
            pltpu.SemaphoreType.DMA,
            pltpu.SemaphoreType.DMA,
            pltpu.SemaphoreType.DMA,
        ],
        compiler_params=pltpu.CompilerParams(use_tc_tiling_on_sc=False),
    )
    def k(dst_hbm, z_hbm, ones_hbm, pa_hbm, pb_hbm, didx, ones_v, acc,
          is0, is1, ss0, ss1):
        c = lax.axis_index("c")
        s = lax.axis_index("s")
        r0 = s * RPS
        pltpu.sync_copy(z_hbm.at[pl.ds(r0, RPS)], acc.at[pl.ds(r0, RPS)])
        pltpu.sync_copy(ones_hbm, ones_v)
        plsc.subcore_barrier()

        cbase = (c * EPC_DEG + s * EPS_DEG) // CH
        NO = NCH_DEG // IW

        def idx_drain(b, sem):
            pltpu.make_async_copy(dst_hbm.at[pl.ds(0, IW)], didx.at[b],
                                  sem).wait()

        def scat_drain(sem):
            pltpu.make_async_copy(z_hbm.at[pl.ds(0, CH)], ones_v, sem).wait()

        pltpu.async_copy(dst_hbm.at[pl.ds(cbase, IW)], didx.at[0], is0)

        @pl.loop(0, NO // 2)
        def _(o2):
            o = o2 * 2
            @pl.when(o2 > 0)
            def _():
                for _ in range(IW):
                    scat_drain(ss1)

            pltpu.async_copy(dst_hbm.at[pl.ds(cbase + (o + 1) * IW, IW)],
                             didx.at[1], is1)
            idx_drain(0, is0)
            for j in range(IW):
                pltpu.async_copy(ones_v, acc.at[didx.at[0].at[j]], ss0,
                                 add=True)

            for _ in range(IW):
                scat_drain(ss0)

            @pl.when(o2 + 1 < NO // 2)
            def _():
                pltpu.async_copy(dst_hbm.at[pl.ds(cbase + (o + 2) * IW, IW)],
                                 didx.at[0], is0)

            idx_drain(1, is1)
            for j in range(IW):
                pltpu.async_copy(ones_v, acc.at[didx.at[1].at[j]], ss1,
                                 add=True)

        for _ in range(IW):
            scat_drain(ss1)
        plsc.subcore_barrier()

        @pl.when(c == 0)
        def _():
            pltpu.sync_copy(acc.at[pl.ds(r0, RPS)], pa_hbm.at[pl.ds(r0, RPS)])

        @pl.when(c == 1)
        def _():
            pltpu.sync_copy(acc.at[pl.ds(r0, RPS)], pb_hbm.at[pl.ds(r0, RPS)])

    return k(dstp.reshape(-1, CH), z16, ones16)


def _sc_message(srcp, dstp, ta, tb, z32):
    mesh = plsc.VectorSubcoreMesh(core_axis_name="c", subcore_axis_name="s",
                                  num_cores=NSC, num_subcores=NSUB)

    @functools.partial(
        pl.kernel,
        out_type=[jax.ShapeDtypeStruct((N_ACC, HH), _f32)] * 2,
        mesh=mesh,
        scratch_types=[
            pltpu.VMEM((IW, CH), jnp.int32),
            pltpu.VMEM((IW, CH), jnp.int32),
            pltpu.VMEM((4, CH, HH), _f32),
            pltpu.VMEM_SHARED((N_ACC, HH), _f32),
            pltpu.SemaphoreType.DMA,
            pltpu.SemaphoreType.DMA,
            pltpu.SemaphoreType.DMA,
            pltpu.SemaphoreType.DMA,
            pltpu.SemaphoreType.DMA,
            pltpu.SemaphoreType.DMA,
            pltpu.SemaphoreType.DMA,
            pltpu.SemaphoreType.DMA,
        ],
        compiler_params=pltpu.CompilerParams(use_tc_tiling_on_sc=False),
    )
    def k(src_hbm, dst_hbm, ta_hbm, tb_hbm, z_hbm, oa_hbm, ob_hbm,
          sidx, didx, rows, acc, gs0, gs1, gs2, gs3, ss0, ss1, ss2, ss3):
        c = lax.axis_index("c")
        s = lax.axis_index("s")
        r0 = s * RPS
        pltpu.sync_copy(z_hbm.at[pl.ds(r0, RPS)], acc.at[pl.ds(r0, RPS)])
        plsc.subcore_barrier()

        gsem = (gs0, gs1, gs2, gs3)
        ssem = (ss0, ss1, ss2, ss3)

        def start_gather(jj, buf, sem):
            @pl.when(c == 0)
            def _():
                pltpu.async_copy(ta_hbm.at[sidx.at[jj]], buf, sem)

            @pl.when(c == 1)
            def _():
                pltpu.async_copy(tb_hbm.at[sidx.at[jj]], buf, sem)

        def drain(buf, sem):
            pltpu.make_async_copy(z_hbm.at[pl.ds(0, CH)], buf, sem).wait()

        sbase8 = s * (EPS_MSG // CH)

        @pl.loop(0, NCH_MSG // IW)
        def _(o):
            ro = sbase8 + o * IW

            @pl.when(o > 0)
            def _():
                for b in range(4):
                    drain(rows.at[b], ssem[b])

            pltpu.sync_copy(src_hbm.at[pl.ds(ro, IW)], sidx)
            pltpu.sync_copy(dst_hbm.at[pl.ds(ro, IW)], didx)

            start_gather(0, rows.at[0], gs0)
            start_gather(1, rows.at[1], gs1)
            for j in range(IW):
                b = j % 4
                if j + 2 < IW:
                    fb = (j + 2) % 4
                    if j >= 2:
                        drain(rows.at[fb], ssem[fb])
                    start_gather(j + 2, rows.at[fb], gsem[fb])
                drain(rows.at[b], gsem[b])
                pltpu.async_copy(rows.at[b], acc.at[didx.at[j]], ssem[b],
                                 add=True)

        for b in range(4):
            drain(rows.at[b], ssem[b])
        plsc.subcore_barrier()

        @pl.when(c == 0)
        def _():
            pltpu.sync_copy(acc.at[pl.ds(r0, RPS)], oa_hbm.at[pl.ds(r0, RPS)])

        @pl.when(c == 1)
        def _():
            pltpu.sync_copy(acc.at[pl.ds(r0, RPS)], ob_hbm.at[pl.ds(r0, RPS)])

    return k(srcp.reshape(-1, CH), dstp.reshape(-1, CH), ta, tb, z32)


def _dinv_of(p0, p1):
    deg = 1.0 + p0[:, 0:1] + p1[:, 0:1]
    return lax.rsqrt(deg)


def _tc_embed(x, emb_W, emb_b2, gcn1_W):
    def body(x_ref, w_ref, b_ref, w1_ref, hw_ref):
        h0 = jnp.dot(x_ref[...], w_ref[...],
                     preferred_element_type=_f32) + b_ref[...]
        hw_ref[...] = jnp.dot(h0, w1_ref[...], preferred_element_type=_f32)

    return pl.pallas_call(
        body,
        grid=(NRB,),
        in_specs=[
            pl.BlockSpec((RB, D), lambda i: (i, 0)),
            pl.BlockSpec((D, H), lambda i: (0, 0)),
            pl.BlockSpec((1, H), lambda i: (0, 0)),
            pl.BlockSpec((H, H), lambda i: (0, 0)),
        ],
        out_specs=pl.BlockSpec((RB, H), lambda i: (i, 0)),
        out_shape=jax.ShapeDtypeStruct((N, H), _f32),
    )(x, emb_W, emb_b2, gcn1_W)


def _tc_scale_split(hw, p0, p1):
    def body(hw_ref, p0_ref, p1_ref, ta_ref, tb_ref):
        dinv = _dinv_of(p0_ref[...], p1_ref[...])
        hws = hw_ref[...] * dinv
        ta_ref[...] = hws[:, :HH]
        tb_ref[...] = hws[:, HH:]

    return pl.pallas_call(
        body,
        grid=(NRB,),
        in_specs=[
            pl.BlockSpec((RB, H), lambda i: (i, 0)),
            pl.BlockSpec((RB, 16), lambda i: (i, 0)),
            pl.BlockSpec((RB, 16), lambda i: (i, 0)),
        ],
        out_specs=[
            pl.BlockSpec((RB, HH), lambda i: (i, 0)),
            pl.BlockSpec((RB, HH), lambda i: (i, 0)),
        ],
        out_shape=[jax.ShapeDtypeStruct((N, HH), _f32)] * 2,
    )(hw, p0, p1)


def _tc_combine_stats(agg_a, agg_b, hw, p0, p1, b2):
    def body(aa_ref, ab_ref, hw_ref, p0_ref, p1_ref, b_ref,
             z_ref, st_ref, acc_ref):
        i = pl.program_id(0)

        @pl.when(i == 0)
        def _():
            acc_ref[...] = jnp.zeros_like(acc_ref)

        dinv = _dinv_of(p0_ref[...], p1_ref[...])
        agg = jnp.concatenate([aa_ref[...], ab_ref[...]], axis=1)
        z = agg * dinv + hw_ref[...] * (dinv * dinv) + b_ref[...]
        z_ref[...] = z
        acc_ref[0:1, :] += jnp.sum(z, axis=0, keepdims=True)
        acc_ref[1:2, :] += jnp.sum(z * z, axis=0, keepdims=True)

        @pl.when(i == NRB - 1)
        def _():
            st_ref[...] = acc_ref[...]

    return pl.pallas_call(
        body,
        grid=(NRB,),
        in_specs=[
            pl.BlockSpec((RB, HH), lambda i: (i, 0)),
            pl.BlockSpec((RB, HH), lambda i: (i, 0)),
            pl.BlockSpec((RB, H), lambda i: (i, 0)),
            pl.BlockSpec((RB, 16), lambda i: (i, 0)),
            pl.BlockSpec((RB, 16), lambda i: (i, 0)),
            pl.BlockSpec((1, H), lambda i: (0, 0)),
        ],
        out_specs=[
            pl.BlockSpec((RB, H), lambda i: (i, 0)),
            pl.BlockSpec((8, H), lambda i: (0, 0)),
        ],
        out_shape=[
            jax.ShapeDtypeStruct((N, H), _f32),
            jax.ShapeDtypeStruct((8, H), _f32),
        ],
        scratch_shapes=[pltpu.VMEM((8, H), _f32)],
    )(agg_a, agg_b, hw, p0, p1, b2)


def _tc_bn_relu_matmul(z, st, g2, b2, W, p0, p1):
    def body(z_ref, st_ref, g_ref, b_ref, w_ref, p0_ref, p1_ref,
             hw_ref, ta_ref, tb_ref):
        mean = st_ref[0:1, :] * (1.0 / N)
        ex2 = st_ref[1:2, :] * (1.0 / N)
        var = ex2 - mean * mean
        inv = lax.rsqrt(var + 1e-5)
        h = jnp.maximum((z_ref[...] - mean) * inv * g_ref[...] + b_ref[...],
                        0.0)
        hw = jnp.dot(h, w_ref[...], preferred_element_type=_f32)
        hw_ref[...] = hw
        dinv = _dinv_of(p0_ref[...], p1_ref[...])
        hws = hw * dinv
        ta_ref[...] = hws[:, :HH]
        tb_ref[...] = hws[:, HH:]

    return pl.pallas_call(
        body,
        grid=(NRB,),
        in_specs=[
            pl.BlockSpec((RB, H), lambda i: (i, 0)),
            pl.BlockSpec((8, H), lambda i: (0, 0)),
            pl.BlockSpec((1, H), lambda i: (0, 0)),
            pl.BlockSpec((1, H), lambda i: (0, 0)),
            pl.BlockSpec((H, H), lambda i: (0, 0)),
            pl.BlockSpec((RB, 16), lambda i: (i, 0)),
            pl.BlockSpec((RB, 16), lambda i: (i, 0)),
        ],
        out_specs=[
            pl.BlockSpec((RB, H), lambda i: (i, 0)),
            pl.BlockSpec((RB, HH), lambda i: (i, 0)),
            pl.BlockSpec((RB, HH), lambda i: (i, 0)),
        ],
        out_shape=[
            jax.ShapeDtypeStruct((N, H), _f32),
            jax.ShapeDtypeStruct((N, HH), _f32),
            jax.ShapeDtypeStruct((N, HH), _f32),
        ],
    )(z, st, g2, b2, W, p0, p1)


def _tc_bn_relu_pool_project(z, st, g2, b2, batch3, fin_W, fin_b2):
    def body(z_ref, st_ref, g_ref, b_ref, bt_ref, fw_ref, fb_ref,
             out_ref, ssum_ref, smax_ref, cnt_ref):
        i = pl.program_id(0)

        @pl.when(i == 0)
        def _():
            ssum_ref[...] = jnp.zeros_like(ssum_ref)
            smax_ref[...] = jnp.zeros_like(smax_ref)
            cnt_ref[...] = jnp.zeros_like(cnt_ref)

        mean = st_ref[0:1, :] * (1.0 / N)
        ex2 = st_ref[1:2, :] * (1.0 / N)
        var = ex2 - mean * mean
        inv = lax.rsqrt(var + 1e-5)
        h = jnp.maximum((z_ref[...] - mean) * inv * g_ref[...] + b_ref[...],
                        0.0)

        bt = bt_ref[0, 0, :]
        onehot = (bt[:, None] ==
                  lax.broadcasted_iota(jnp.int32, (PB, G), 1)).astype(_f32)
        ssum_ref[...] += lax.dot_general(
            onehot, h, (((0,), (0,)), ((), ())), preferred_element_type=_f32)
        cnt_ref[:, 0:1] += lax.dot_general(
            onehot, jnp.ones((PB, 1), _f32), (((0,), (0,)), ((), ())),
            preferred_element_type=_f32)
        g_lo = jnp.min(bt)
        g_hi = jnp.max(bt)

        def _mbody(g, carry):
            col = (bt == g).astype(_f32)[:, None]
            m = jnp.max(h * col, axis=0, keepdims=True)
            cur = smax_ref[pl.ds(g, 1), :]
            smax_ref[pl.ds(g, 1), :] = jnp.maximum(cur, m)
            return carry

        lax.fori_loop(g_lo, g_hi + 1, _mbody, 0)

        @pl.when(i == NPB - 1)
        def _():
            cnt = cnt_ref[:, 0:1]
            meanp = ssum_ref[...] / jnp.maximum(cnt, 1.0)
            pooled = jnp.concatenate([smax_ref[...], meanp], axis=1)
            out_ref[...] = jnp.dot(pooled, fw_ref[...],
                                   preferred_element_type=_f32) + fb_ref[...]

    return pl.pallas_call(
        body,
        grid=(NPB,),
        in_specs=[
            pl.BlockSpec((PB, H), lambda i: (i, 0)),
            pl.BlockSpec((8, H), lambda i: (0, 0)),
            pl.BlockSpec((1, H), lambda i: (0, 0)),
            pl.BlockSpec((1, H), lambda i: (0, 0)),
            pl.BlockSpec((1, 1, PB), lambda i: (i, 0, 0)),
            pl.BlockSpec((2 * H, D), lambda i: (0, 0)),
            pl.BlockSpec((1, D), lambda i: (0, 0)),
        ],
        out_specs=pl.BlockSpec((G, D), lambda i: (0, 0)),
        out_shape=jax.ShapeDtypeStruct((G, D), _f32),
        scratch_shapes=[
            pltpu.VMEM((G, H), _f32),
            pltpu.VMEM((G, H), _f32),
            pltpu.VMEM((G, 128), _f32),
        ],
    )(z, st, g2, b2, batch3, fin_W, fin_b2)


def kernel(x, edge_index, batch, emb_W, emb_b, gcn1_W, gcn1_b, bn1_g, bn1_b,
           gcn2_W, gcn2_b, bn2_g, bn2_b, fin_W, fin_b):
    pad = E_PAD - E
    srcp = jnp.concatenate([edge_index[0], jnp.zeros((pad,), jnp.int32)])
    dstp = jnp.concatenate(
        [edge_index[1], jnp.full((pad,), DUMP, jnp.int32)])
    z16 = jnp.zeros((N_ACC, 16), _f32)
    z32 = jnp.zeros((N_ACC, HH), _f32)
    ones16 = jnp.ones((CH, 16), _f32)
    batch3 = batch.reshape(NPB, 1, PB)

    emb_b2 = emb_b.reshape(1, H)
    b1 = gcn1_b.reshape(1, H)
    b2 = gcn2_b.reshape(1, H)
    g1 = bn1_g.reshape(1, H)
    be1 = bn1_b.reshape(1, H)
    g2 = bn2_g.reshape(1, H)
    be2 = bn2_b.reshape(1, H)
    fb2 = fin_b.reshape(1, D)

    p0, p1 = _sc_degree(dstp, z16, ones16)
    hw1 = _tc_embed(x, emb_W, emb_b2, gcn1_W)

    t1a, t1b = _tc_scale_split(hw1, p0, p1)
    agg1a, agg1b = _sc_message(srcp, dstp, t1a, t1b, z32)
    z1, st1 = _tc_combine_stats(agg1a, agg1b, hw1, p0, p1, b1)

    hw2, t2a, t2b = _tc_bn_relu_matmul(z1, st1, g1, be1, gcn2_W, p0, p1)
    agg2a, agg2b = _sc_message(srcp, dstp, t2a, t2b, z32)
    z2, st2 = _tc_combine_stats(agg2a, agg2b, hw2, p0, p1, b2)

    return _tc_bn_relu_pool_project(z2, st2, g2, be2, batch3, fin_W, fb2)


# speedup vs baseline: 18.1124x; 1.0406x over previous
import functools

import jax
import jax.numpy as jnp
from jax import lax
from jax.experimental import pallas as pl
from jax.experimental.pallas import tpu as pltpu
from jax.experimental.pallas import tpu_sc as plsc

N = 50000
E = 800000
D = 128
H = 64
HH = 32
G = 64

NSC = 2
NSUB = 16
CH = 128
IW = 10
GW = 4

N_ACC = 50176
RPS = N_ACC // NSUB
DUMP = N

E_PAD = 819200
EPS_MSG = E_PAD // NSUB
NCH_MSG = EPS_MSG // CH
EPC_DEG = E_PAD // NSC
EPS_DEG = EPC_DEG // NSUB
NCH_DEG = EPS_DEG // CH

RB = 1000
NRB = N // RB
PB = 1000
NPB = N // PB

_f32 = jnp.float32


def _sc_degree(dstp, z16, ones16):
    mesh = plsc.VectorSubcoreMesh(core_axis_name="c", subcore_axis_name="s",
                                  num_cores=NSC, num_subcores=NSUB)

    @functools.partial(
        pl.kernel,
        out_type=[jax.ShapeDtypeStruct((N_ACC, 16), _f32)] * 2,
        mesh=mesh,
        scratch_types=[
            pltpu.VMEM((2, IW, CH), jnp.int32),
            pltpu.VMEM((CH, 16), _f32),
            pltpu.VMEM_SHARED((N_ACC, 16), _f32),
            pltpu.SemaphoreType.DMA,
            pltpu.SemaphoreType.DMA,
            pltpu.SemaphoreType.DMA,
            pltpu.SemaphoreType.DMA,
        ],
        compiler_params=pltpu.CompilerParams(use_tc_tiling_on_sc=False),
    )
    def k(dst_hbm, z_hbm, ones_hbm, pa_hbm, pb_hbm, didx, ones_v, acc,
          is0, is1, ss0, ss1):
        c = lax.axis_index("c")
        s = lax.axis_index("s")
        r0 = s * RPS
        pltpu.sync_copy(z_hbm.at[pl.ds(r0, RPS)], acc.at[pl.ds(r0, RPS)])
        pltpu.sync_copy(ones_hbm, ones_v)
        plsc.subcore_barrier()

        cbase = (c * EPC_DEG + s * EPS_DEG) // CH
        NO = NCH_DEG // IW

        def idx_drain(b, sem):
            pltpu.make_async_copy(dst_hbm.at[pl.ds(0, IW)], didx.at[b],
                                  sem).wait()

        def scat_drain(sem):
            pltpu.make_async_copy(z_hbm.at[pl.ds(0, CH)], ones_v, sem).wait()

        pltpu.async_copy(dst_hbm.at[pl.ds(cbase, IW)], didx.at[0], is0)

        @pl.loop(0, NO // 2)
        def _(o2):
            o = o2 * 2
            @pl.when(o2 > 0)
            def _():
                for _ in range(IW):
                    scat_drain(ss1)

            pltpu.async_copy(dst_hbm.at[pl.ds(cbase + (o + 1) * IW, IW)],
                             didx.at[1], is1)
            idx_drain(0, is0)
            for j in range(IW):
                pltpu.async_copy(ones_v, acc.at[didx.at[0].at[j]], ss0,
                                 add=True)

            for _ in range(IW):
                scat_drain(ss0)

            @pl.when(o2 + 1 < NO // 2)
            def _():
                pltpu.async_copy(dst_hbm.at[pl.ds(cbase + (o + 2) * IW, IW)],
                                 didx.at[0], is0)

            idx_drain(1, is1)
            for j in range(IW):
                pltpu.async_copy(ones_v, acc.at[didx.at[1].at[j]], ss1,
                                 add=True)

        for _ in range(IW):
            scat_drain(ss1)
        plsc.subcore_barrier()

        @pl.when(c == 0)
        def _():
            pltpu.sync_copy(acc.at[pl.ds(r0, RPS)], pa_hbm.at[pl.ds(r0, RPS)])

        @pl.when(c == 1)
        def _():
            pltpu.sync_copy(acc.at[pl.ds(r0, RPS)], pb_hbm.at[pl.ds(r0, RPS)])

    return k(dstp.reshape(-1, CH), z16, ones16)


def _sc_message(srcp, dstp, ta, tb, z32):
    mesh = plsc.VectorSubcoreMesh(core_axis_name="c", subcore_axis_name="s",
                                  num_cores=NSC, num_subcores=NSUB)

    @functools.partial(
        pl.kernel,
        out_type=[jax.ShapeDtypeStruct((N_ACC, HH), _f32)] * 2,
        mesh=mesh,
        scratch_types=[
            pltpu.VMEM((GW, IW, CH), jnp.int32),
            pltpu.VMEM((GW, IW, CH), jnp.int32),
            pltpu.VMEM((4, CH, HH), _f32),
            pltpu.VMEM_SHARED((N_ACC, HH), _f32),
            pltpu.SemaphoreType.DMA,
            pltpu.SemaphoreType.DMA,
            pltpu.SemaphoreType.DMA,
            pltpu.SemaphoreType.DMA,
            pltpu.SemaphoreType.DMA,
            pltpu.SemaphoreType.DMA,
            pltpu.SemaphoreType.DMA,
            pltpu.SemaphoreType.DMA,
        ],
        compiler_params=pltpu.CompilerParams(use_tc_tiling_on_sc=False),
    )
    def k(src_hbm, dst_hbm, ta_hbm, tb_hbm, z_hbm, oa_hbm, ob_hbm,
          sidx, didx, rows, acc, gs0, gs1, gs2, gs3, ss0, ss1, ss2, ss3):
        c = lax.axis_index("c")
        s = lax.axis_index("s")
        r0 = s * RPS
        pltpu.sync_copy(z_hbm.at[pl.ds(r0, RPS)], acc.at[pl.ds(r0, RPS)])
        plsc.subcore_barrier()

        gsem = (gs0, gs1, gs2, gs3)
        ssem = (ss0, ss1, ss2, ss3)

        def start_gather(q, jj, buf, sem):
            @pl.when(c == 0)
            def _():
                pltpu.async_copy(ta_hbm.at[sidx.at[q].at[jj]], buf, sem)

            @pl.when(c == 1)
            def _():
                pltpu.async_copy(tb_hbm.at[sidx.at[q].at[jj]], buf, sem)

        def drain(buf, sem):
            pltpu.make_async_copy(z_hbm.at[pl.ds(0, CH)], buf, sem).wait()

        sbase = s * (NCH_MSG // IW)

        @pl.loop(0, NCH_MSG // IW)
        def _(o):
            q = lax.rem(o, GW)
            at_group = q == 0

            @pl.when(jnp.logical_and(at_group, o > 0))
            def _():
                for b in range(4):
                    drain(rows.at[b], ssem[b])

            @pl.when(at_group)
            def _():
                pltpu.sync_copy(src_hbm.at[pl.ds(sbase + o, GW)], sidx)
                pltpu.sync_copy(dst_hbm.at[pl.ds(sbase + o, GW)], didx)

            @pl.when(jnp.logical_not(at_group))
            def _():
                drain(rows.at[0], ss0)
                drain(rows.at[1], ss1)

            start_gather(q, 0, rows.at[0], gs0)
            start_gather(q, 1, rows.at[1], gs1)
            for j in range(IW):
                b = j % 4
                if j + 2 < IW:
                    fb = (j + 2) % 4
                    if j >= 2:
                        drain(rows.at[fb], ssem[fb])
                    else:
                        @pl.when(jnp.logical_not(at_group))
                        def _():
                            drain(rows.at[fb], ssem[fb])
                    start_gather(q, j + 2, rows.at[fb], gsem[fb])
                drain(rows.at[b], gsem[b])
                pltpu.async_copy(rows.at[b], acc.at[didx.at[q].at[j]],
                                 ssem[b], add=True)

        for b in range(4):
            drain(rows.at[b], ssem[b])
        plsc.subcore_barrier()

        @pl.when(c == 0)
        def _():
            pltpu.sync_copy(acc.at[pl.ds(r0, RPS)], oa_hbm.at[pl.ds(r0, RPS)])

        @pl.when(c == 1)
        def _():
            pltpu.sync_copy(acc.at[pl.ds(r0, RPS)], ob_hbm.at[pl.ds(r0, RPS)])

    return k(srcp.reshape(-1, IW, CH), dstp.reshape(-1, IW, CH), ta, tb, z32)


def _dinv_of(p0, p1):
    deg = 1.0 + p0[:, 0:1] + p1[:, 0:1]
    return lax.rsqrt(deg)


def _tc_embed(x, emb_W, emb_b2, gcn1_W):
    def body(x_ref, w_ref, b_ref, w1_ref, hw_ref):
        h0 = jnp.dot(x_ref[...], w_ref[...],
                     preferred_element_type=_f32) + b_ref[...]
        hw_ref[...] = jnp.dot(h0, w1_ref[...], preferred_element_type=_f32)

    return pl.pallas_call(
        body,
        grid=(NRB,),
        in_specs=[
            pl.BlockSpec((RB, D), lambda i: (i, 0)),
            pl.BlockSpec((D, H), lambda i: (0, 0)),
            pl.BlockSpec((1, H), lambda i: (0, 0)),
            pl.BlockSpec((H, H), lambda i: (0, 0)),
        ],
        out_specs=pl.BlockSpec((RB, H), lambda i: (i, 0)),
        out_shape=jax.ShapeDtypeStruct((N, H), _f32),
    )(x, emb_W, emb_b2, gcn1_W)


def _tc_scale_split(hw, p0, p1):
    def body(hw_ref, p0_ref, p1_ref, ta_ref, tb_ref):
        dinv = _dinv_of(p0_ref[...], p1_ref[...])
        hws = hw_ref[...] * dinv
        ta_ref[...] = hws[:, :HH]
        tb_ref[...] = hws[:, HH:]

    return pl.pallas_call(
        body,
        grid=(NRB,),
        in_specs=[
            pl.BlockSpec((RB, H), lambda i: (i, 0)),
            pl.BlockSpec((RB, 16), lambda i: (i, 0)),
            pl.BlockSpec((RB, 16), lambda i: (i, 0)),
        ],
        out_specs=[
            pl.BlockSpec((RB, HH), lambda i: (i, 0)),
            pl.BlockSpec((RB, HH), lambda i: (i, 0)),
        ],
        out_shape=[jax.ShapeDtypeStruct((N, HH), _f32)] * 2,
    )(hw, p0, p1)


def _tc_combine_stats(agg_a, agg_b, hw, p0, p1, b2):
    def body(aa_ref, ab_ref, hw_ref, p0_ref, p1_ref, b_ref,
             z_ref, st_ref, acc_ref):
        i = pl.program_id(0)

        @pl.when(i == 0)
        def _():
            acc_ref[...] = jnp.zeros_like(acc_ref)

        dinv = _dinv_of(p0_ref[...], p1_ref[...])
        agg = jnp.concatenate([aa_ref[...], ab_ref[...]], axis=1)
        z = agg * dinv + hw_ref[...] * (dinv * dinv) + b_ref[...]
        z_ref[...] = z
        acc_ref[0:1, :] += jnp.sum(z, axis=0, keepdims=True)
        acc_ref[1:2, :] += jnp.sum(z * z, axis=0, keepdims=True)

        @pl.when(i == NRB - 1)
        def _():
            st_ref[...] = acc_ref[...]

    return pl.pallas_call(
        body,
        grid=(NRB,),
        in_specs=[
            pl.BlockSpec((RB, HH), lambda i: (i, 0)),
            pl.BlockSpec((RB, HH), lambda i: (i, 0)),
            pl.BlockSpec((RB, H), lambda i: (i, 0)),
            pl.BlockSpec((RB, 16), lambda i: (i, 0)),
            pl.BlockSpec((RB, 16), lambda i: (i, 0)),
            pl.BlockSpec((1, H), lambda i: (0, 0)),
        ],
        out_specs=[
            pl.BlockSpec((RB, H), lambda i: (i, 0)),
            pl.BlockSpec((8, H), lambda i: (0, 0)),
        ],
        out_shape=[
            jax.ShapeDtypeStruct((N, H), _f32),
            jax.ShapeDtypeStruct((8, H), _f32),
        ],
        scratch_shapes=[pltpu.VMEM((8, H), _f32)],
    )(agg_a, agg_b, hw, p0, p1, b2)


def _tc_bn_relu_matmul(z, st, g2, b2, W, p0, p1):
    def body(z_ref, st_ref, g_ref, b_ref, w_ref, p0_ref, p1_ref,
             hw_ref, ta_ref, tb_ref):
        mean = st_ref[0:1, :] * (1.0 / N)
        ex2 = st_ref[1:2, :] * (1.0 / N)
        var = ex2 - mean * mean
        inv = lax.rsqrt(var + 1e-5)
        h = jnp.maximum((z_ref[...] - mean) * inv * g_ref[...] + b_ref[...],
                        0.0)
        hw = jnp.dot(h, w_ref[...], preferred_element_type=_f32)
        hw_ref[...] = hw
        dinv = _dinv_of(p0_ref[...], p1_ref[...])
        hws = hw * dinv
        ta_ref[...] = hws[:, :HH]
        tb_ref[...] = hws[:, HH:]

    return pl.pallas_call(
        body,
        grid=(NRB,),
        in_specs=[
            pl.BlockSpec((RB, H), lambda i: (i, 0)),
            pl.BlockSpec((8, H), lambda i: (0, 0)),
            pl.BlockSpec((1, H), lambda i: (0, 0)),
            pl.BlockSpec((1, H), lambda i: (0, 0)),
            pl.BlockSpec((H, H), lambda i: (0, 0)),
            pl.BlockSpec((RB, 16), lambda i: (i, 0)),
            pl.BlockSpec((RB, 16), lambda i: (i, 0)),
        ],
        out_specs=[
            pl.BlockSpec((RB, H), lambda i: (i, 0)),
            pl.BlockSpec((RB, HH), lambda i: (i, 0)),
            pl.BlockSpec((RB, HH), lambda i: (i, 0)),
        ],
        out_shape=[
            jax.ShapeDtypeStruct((N, H), _f32),
            jax.ShapeDtypeStruct((N, HH), _f32),
            jax.ShapeDtypeStruct((N, HH), _f32),
        ],
    )(z, st, g2, b2, W, p0, p1)


def _tc_bn_relu_pool_project(z, st, g2, b2, batch3, fin_W, fin_b2):
    def body(z_ref, st_ref, g_ref, b_ref, bt_ref, fw_ref, fb_ref,
             out_ref, ssum_ref, smax_ref, cnt_ref):
        i = pl.program_id(0)

        @pl.when(i == 0)
        def _():
            ssum_ref[...] = jnp.zeros_like(ssum_ref)
            smax_ref[...] = jnp.zeros_like(smax_ref)
            cnt_ref[...] = jnp.zeros_like(cnt_ref)

        mean = st_ref[0:1, :] * (1.0 / N)
        ex2 = st_ref[1:2, :] * (1.0 / N)
        var = ex2 - mean * mean
        inv = lax.rsqrt(var + 1e-5)
        h = jnp.maximum((z_ref[...] - mean) * inv * g_ref[...] + b_ref[...],
                        0.0)

        bt = bt_ref[0, 0, :]
        onehot = (bt[:, None] ==
                  lax.broadcasted_iota(jnp.int32, (PB, G), 1)).astype(_f32)
        ssum_ref[...] += lax.dot_general(
            onehot, h, (((0,), (0,)), ((), ())), preferred_element_type=_f32)
        cnt_ref[:, 0:1] += lax.dot_general(
            onehot, jnp.ones((PB, 1), _f32), (((0,), (0,)), ((), ())),
            preferred_element_type=_f32)
        g_lo = jnp.min(bt)
        g_hi = jnp.max(bt)

        def _mbody(g, carry):
            col = (bt == g).astype(_f32)[:, None]
            m = jnp.max(h * col, axis=0, keepdims=True)
            cur = smax_ref[pl.ds(g, 1), :]
            smax_ref[pl.ds(g, 1), :] = jnp.maximum(cur, m)
            return carry

        lax.fori_loop(g_lo, g_hi + 1, _mbody, 0)

        @pl.when(i == NPB - 1)
        def _():
            cnt = cnt_ref[:, 0:1]
            meanp = ssum_ref[...] / jnp.maximum(cnt, 1.0)
            pooled = jnp.concatenate([smax_ref[...], meanp], axis=1)
            out_ref[...] = jnp.dot(pooled, fw_ref[...],
                                   preferred_element_type=_f32) + fb_ref[...]

    return pl.pallas_call(
        body,
        grid=(NPB,),
        in_specs=[
            pl.BlockSpec((PB, H), lambda i: (i, 0)),
            pl.BlockSpec((8, H), lambda i: (0, 0)),
            pl.BlockSpec((1, H), lambda i: (0, 0)),
            pl.BlockSpec((1, H), lambda i: (0, 0)),
            pl.BlockSpec((1, 1, PB), lambda i: (i, 0, 0)),
            pl.BlockSpec((2 * H, D), lambda i: (0, 0)),
            pl.BlockSpec((1, D), lambda i: (0, 0)),
        ],
        out_specs=pl.BlockSpec((G, D), lambda i: (0, 0)),
        out_shape=jax.ShapeDtypeStruct((G, D), _f32),
        scratch_shapes=[
            pltpu.VMEM((G, H), _f32),
            pltpu.VMEM((G, H), _f32),
            pltpu.VMEM((G, 128), _f32),
        ],
    )(z, st, g2, b2, batch3, fin_W, fin_b2)


def kernel(x, edge_index, batch, emb_W, emb_b, gcn1_W, gcn1_b, bn1_g, bn1_b,
           gcn2_W, gcn2_b, bn2_g, bn2_b, fin_W, fin_b):
    pad = E_PAD - E
    srcp = jnp.concatenate([edge_index[0], jnp.zeros((pad,), jnp.int32)])
    dstp = jnp.concatenate(
        [edge_index[1], jnp.full((pad,), DUMP, jnp.int32)])
    z16 = jnp.zeros((N_ACC, 16), _f32)
    z32 = jnp.zeros((N_ACC, HH), _f32)
    ones16 = jnp.ones((CH, 16), _f32)
    batch3 = batch.reshape(NPB, 1, PB)

    emb_b2 = emb_b.reshape(1, H)
    b1 = gcn1_b.reshape(1, H)
    b2 = gcn2_b.reshape(1, H)
    g1 = bn1_g.reshape(1, H)
    be1 = bn1_b.reshape(1, H)
    g2 = bn2_g.reshape(1, H)
    be2 = bn2_b.reshape(1, H)
    fb2 = fin_b.reshape(1, D)

    p0, p1 = _sc_degree(dstp, z16, ones16)
    hw1 = _tc_embed(x, emb_W, emb_b2, gcn1_W)

    t1a, t1b = _tc_scale_split(hw1, p0, p1)
    agg1a, agg1b = _sc_message(srcp, dstp, t1a, t1b, z32)
    z1, st1 = _tc_combine_stats(agg1a, agg1b, hw1, p0, p1, b1)

    hw2, t2a, t2b = _tc_bn_relu_matmul(z1, st1, g1, be1, gcn2_W, p0, p1)
    agg2a, agg2b = _sc_message(srcp, dstp, t2a, t2b, z32)
    z2, st2 = _tc_combine_stats(agg2a, agg2b, hw2, p0, p1, b2)

    return _tc_bn_relu_pool_project(z2, st2, g2, be2, batch3, fin_W, fb2)
